# baseline (speedup 1.0000x reference)
import jax
import jax.numpy as jnp
from jax import lax
from jax.experimental import pallas as pl
from jax.experimental.pallas import tpu as pltpu
from jax.experimental.pallas import tpu_sc as plsc

E = 320000
N = 10000
D = 128

NC = 2
NS = 16
NW = NC * NS
EPW = E // NW
CHUNK = 64
NCHUNK = EPW // CHUNK
REM = EPW - NCHUNK * CHUNK
ZR = 624
ZTAIL = N - NS * ZR


NBUF = 3
MAIN = (NCHUNK // NBUF) * NBUF


def _sc_scatter(*refs):
    (pe_hbm, w_hbm, dst_hbm, zero_hbm, part_hbm, peout_hbm) = refs[:6]
    r = list(refs[6:])
    ebufs = [r.pop(0) for _ in range(NBUF)]
    obufs = [r.pop(0) for _ in range(NBUF)]
    wrs = [r.pop(0) for _ in range(NBUF)]
    irs = [r.pop(0) for _ in range(NBUF)]
    ir_rem = r.pop(0)
    acc = r.pop(0)
    in_sems = [r.pop(0) for _ in range(NBUF)]
    out_sems = [r.pop(0) for _ in range(NBUF)]
    wb_sems = [r.pop(0) for _ in range(NBUF)]
    c = lax.axis_index("c")
    s = lax.axis_index("s")
    wid = c * NS + s
    ebase = wid * EPW

    pltpu.sync_copy(zero_hbm.at[pl.ds(s * ZR, ZR)], acc.at[pl.ds(s * ZR, ZR)])

    @pl.when(s == 0)
    def _zero_tail():
        pltpu.sync_copy(zero_hbm.at[pl.ds(NS * ZR, ZTAIL)],
                        acc.at[pl.ds(NS * ZR, ZTAIL)])

    plsc.subcore_barrier()

    def _in_start(b, j):
        base = ebase + j * CHUNK
        pltpu.async_copy(pe_hbm.at[pl.ds(base, CHUNK), :], ebufs[b],
                         in_sems[b])
        pltpu.async_copy(w_hbm.at[pl.ds(base, CHUNK)], wrs[b], in_sems[b])
        pltpu.async_copy(dst_hbm.at[pl.ds(base, CHUNK)], irs[b], in_sems[b])

    def _in_wait(b, j):
        base = ebase + j * CHUNK
        pltpu.make_async_copy(pe_hbm.at[pl.ds(base, CHUNK), :], ebufs[b],
                              in_sems[b]).wait()
        pltpu.make_async_copy(w_hbm.at[pl.ds(base, CHUNK)], wrs[b],
                              in_sems[b]).wait()
        pltpu.make_async_copy(dst_hbm.at[pl.ds(base, CHUNK)], irs[b],
                              in_sems[b]).wait()

    def _out_wait(b):
        pltpu.make_async_copy(obufs[b], acc.at[irs[b]], out_sems[b]).wait()

    def _wb_start(b, j):
        pltpu.async_copy(ebufs[b],
                         peout_hbm.at[pl.ds(ebase + j * CHUNK, CHUNK), :],
                         wb_sems[b])

    def _wb_wait(b, j):
        pltpu.make_async_copy(ebufs[b],
                              peout_hbm.at[pl.ds(ebase + j * CHUNK, CHUNK), :],
                              wb_sems[b]).wait()

    for b in range(NBUF):
        _in_start(b, b)

    def outer_body(go, _):
        for b in range(NBUF):
            j = go * NBUF + b
            _in_wait(b, j)

            _wb_start(b, j)

            @plsc.parallel_loop(0, CHUNK, unroll=4)
            def _rows(r):
                wspl = plsc.load_gather(
                    wrs[b], [jnp.broadcast_to(r, (16,)).astype(jnp.int32)])
                eb = ebufs[b]
                ob = obufs[b]
                for d in range(D // 16):
                    sl = pl.ds(d * 16, 16)
                    ob[r, sl] = eb[r, sl] * wspl

            pltpu.async_copy(obufs[b], acc.at[irs[b]], out_sems[b], add=True)

            bp = (b - 1) % NBUF

            def _drain_refill():
                _wb_wait(bp, j - 1)
                _out_wait(bp)

                @pl.when(j + NBUF - 1 <= MAIN - 1)
                def _refill():
                    _in_start(bp, j + NBUF - 1)

            if b == 0:
                pl.when(go > 0)(_drain_refill)
            else:
                _drain_refill()
        return 0

    lax.fori_loop(0, MAIN // NBUF, outer_body, 0)

    _wb_wait(NBUF - 1, MAIN - 1)
    _out_wait(NBUF - 1)

    for t in range(NCHUNK - MAIN):
        j = MAIN + t
        base = ebase + j * CHUNK
        pltpu.sync_copy(pe_hbm.at[pl.ds(base, CHUNK), :], ebufs[t])
        pltpu.sync_copy(w_hbm.at[pl.ds(base, CHUNK)], wrs[t])
        pltpu.sync_copy(dst_hbm.at[pl.ds(base, CHUNK)], irs[t])
        pltpu.sync_copy(ebufs[t], peout_hbm.at[pl.ds(base, CHUNK), :])

        @plsc.parallel_loop(0, CHUNK, unroll=4)
        def _rows_tail(r):
            wspl = plsc.load_gather(
                wrs[t], [jnp.broadcast_to(r, (16,)).astype(jnp.int32)])
            eb = ebufs[t]
            ob = obufs[t]
            for d in range(D // 16):
                sl = pl.ds(d * 16, 16)
                ob[r, sl] = eb[r, sl] * wspl

        pltpu.sync_copy(obufs[t], acc.at[irs[t]], add=True)

    if REM > 0:
        rbase = ebase + NCHUNK * CHUNK
        pltpu.sync_copy(pe_hbm.at[pl.ds(rbase, REM), :],
                        ebufs[0].at[pl.ds(0, REM)])
        pltpu.sync_copy(w_hbm.at[pl.ds(rbase, REM)],
                        wrs[0].at[pl.ds(0, REM)])
        pltpu.sync_copy(dst_hbm.at[pl.ds(rbase, REM)], ir_rem)
        pltpu.sync_copy(ebufs[0].at[pl.ds(0, REM)],
                        peout_hbm.at[pl.ds(rbase, REM), :])

        @plsc.parallel_loop(0, REM, unroll=4)
        def _rows_rem(r):
            wspl = plsc.load_gather(
                wrs[0], [jnp.broadcast_to(r, (16,)).astype(jnp.int32)])
            for d in range(D // 16):
                sl = pl.ds(d * 16, 16)
                obufs[0][r, sl] = ebufs[0][r, sl] * wspl

        pltpu.sync_copy(obufs[0].at[pl.ds(0, REM)], acc.at[ir_rem], add=True)

    plsc.subcore_barrier()

    pltpu.sync_copy(acc.at[pl.ds(s * ZR, ZR)],
                    part_hbm.at[c, pl.ds(s * ZR, ZR)])

    @pl.when(s == 0)
    def _read_tail():
        pltpu.sync_copy(acc.at[pl.ds(NS * ZR, ZTAIL)],
                        part_hbm.at[c, pl.ds(NS * ZR, ZTAIL)])


@jax.jit
def _sc_call(pe, w, dst, zeros_nd):
    mesh = plsc.VectorSubcoreMesh(core_axis_name="c", subcore_axis_name="s")
    return pl.kernel(
        _sc_scatter,
        out_type=(
            jax.ShapeDtypeStruct((NC, N, D), jnp.float32),
            jax.ShapeDtypeStruct((E, D), jnp.float32),
        ),
        mesh=mesh,
        compiler_params=pltpu.CompilerParams(needs_layout_passes=False),
        scratch_types=(
            [pltpu.VMEM((CHUNK, D), jnp.float32)] * NBUF
            + [pltpu.VMEM((CHUNK, D), jnp.float32)] * NBUF
            + [pltpu.VMEM((CHUNK,), jnp.float32)] * NBUF
            + [pltpu.VMEM((CHUNK,), jnp.int32)] * NBUF
            + [pltpu.VMEM((max(REM, 8),), jnp.int32)]
            + [pltpu.VMEM_SHARED((N, D), jnp.float32)]
            + [pltpu.SemaphoreType.DMA] * (3 * NBUF)
        ),
    )(pe, w, dst, zeros_nd)


def _combine_body(p0_ref, p1_ref, ns_ref, norm_ref, recall_ref):
    recall = p0_ref[...] + p1_ref[...]
    recall_ref[...] = recall
    norm_ref[...] = recall / ns_ref[...]


@jax.jit
def _combine(parts, ns):
    blk = 5000
    grid = (N // blk,)
    return pl.pallas_call(
        _combine_body,
        grid=grid,
        in_specs=[
            pl.BlockSpec((blk, D), lambda i: (i, 0)),
            pl.BlockSpec((blk, D), lambda i: (i, 0)),
            pl.BlockSpec((blk, 1), lambda i: (i, 0)),
        ],
        out_specs=[
            pl.BlockSpec((blk, D), lambda i: (i, 0)),
            pl.BlockSpec((blk, D), lambda i: (i, 0)),
        ],
        out_shape=[
            jax.ShapeDtypeStruct((N, D), jnp.float32),
            jax.ShapeDtypeStruct((N, D), jnp.float32),
        ],
    )(parts[0], parts[1], ns)


def kernel(poss_edge, weights, neighbours_sum, edges):
    dst = edges[:, 0]
    zeros_nd = jnp.zeros((N, D), jnp.float32)
    parts, pe_out = _sc_call(poss_edge, weights, dst, zeros_nd)
    norm, recall = _combine(parts, neighbours_sum)
    return (norm, pe_out, recall)


# speedup vs baseline: 1.0642x; 1.0642x over previous
import jax
import jax.numpy as jnp
from jax import lax
from jax.experimental import pallas as pl
from jax.experimental.pallas import tpu as pltpu
from jax.experimental.pallas import tpu_sc as plsc

E = 320000
N = 10000
D = 128

NC = 2
NS = 16
NW = NC * NS
EPW = E // NW
CHUNK = 48
NCHUNK = EPW // CHUNK
REM = EPW - NCHUNK * CHUNK
ZR = 624
ZTAIL = N - NS * ZR


NBUF = 4
MAIN = (NCHUNK // NBUF) * NBUF


def _sc_scatter(*refs):
    (pe_hbm, w_hbm, dst_hbm, zero_hbm, part_hbm, peout_hbm) = refs[:6]
    r = list(refs[6:])
    ebufs = [r.pop(0) for _ in range(NBUF)]
    obufs = [r.pop(0) for _ in range(NBUF)]
    wrs = [r.pop(0) for _ in range(NBUF)]
    irs = [r.pop(0) for _ in range(NBUF)]
    ir_rem = r.pop(0)
    acc = r.pop(0)
    in_sems = [r.pop(0) for _ in range(NBUF)]
    out_sems = [r.pop(0) for _ in range(NBUF)]
    wb_sems = [r.pop(0) for _ in range(NBUF)]
    c = lax.axis_index("c")
    s = lax.axis_index("s")
    wid = c * NS + s
    ebase = wid * EPW

    pltpu.sync_copy(zero_hbm.at[pl.ds(s * ZR, ZR)], acc.at[pl.ds(s * ZR, ZR)])

    @pl.when(s == 0)
    def _zero_tail():
        pltpu.sync_copy(zero_hbm.at[pl.ds(NS * ZR, ZTAIL)],
                        acc.at[pl.ds(NS * ZR, ZTAIL)])

    plsc.subcore_barrier()

    def _in_start(b, j):
        base = ebase + j * CHUNK
        pltpu.async_copy(pe_hbm.at[pl.ds(base, CHUNK), :], ebufs[b],
                         in_sems[b])
        pltpu.async_copy(w_hbm.at[pl.ds(base, CHUNK)], wrs[b], in_sems[b])
        pltpu.async_copy(dst_hbm.at[pl.ds(base, CHUNK)], irs[b], in_sems[b])

    def _in_wait(b, j):
        base = ebase + j * CHUNK
        pltpu.make_async_copy(pe_hbm.at[pl.ds(base, CHUNK), :], ebufs[b],
                              in_sems[b]).wait()
        pltpu.make_async_copy(w_hbm.at[pl.ds(base, CHUNK)], wrs[b],
                              in_sems[b]).wait()
        pltpu.make_async_copy(dst_hbm.at[pl.ds(base, CHUNK)], irs[b],
                              in_sems[b]).wait()

    def _out_wait(b):
        pltpu.make_async_copy(obufs[b], acc.at[irs[b]], out_sems[b]).wait()

    def _wb_start(b, j):
        pltpu.async_copy(ebufs[b],
                         peout_hbm.at[pl.ds(ebase + j * CHUNK, CHUNK), :],
                         wb_sems[b])

    def _wb_wait(b, j):
        pltpu.make_async_copy(ebufs[b],
                              peout_hbm.at[pl.ds(ebase + j * CHUNK, CHUNK), :],
                              wb_sems[b]).wait()

    for b in range(NBUF):
        _in_start(b, b)

    def outer_body(go, _):
        for b in range(NBUF):
            j = go * NBUF + b
            _in_wait(b, j)

            _wb_start(b, j)

            @plsc.parallel_loop(0, CHUNK, unroll=4)
            def _rows(r):
                wspl = plsc.load_gather(
                    wrs[b], [jnp.broadcast_to(r, (16,)).astype(jnp.int32)])
                eb = ebufs[b]
                ob = obufs[b]
                for d in range(D // 16):
                    sl = pl.ds(d * 16, 16)
                    ob[r, sl] = eb[r, sl] * wspl

            pltpu.async_copy(obufs[b], acc.at[irs[b]], out_sems[b], add=True)

            bp = (b - 1) % NBUF

            def _drain_refill():
                _wb_wait(bp, j - 1)
                _out_wait(bp)

                @pl.when(j + NBUF - 1 <= MAIN - 1)
                def _refill():
                    _in_start(bp, j + NBUF - 1)

            if b == 0:
                pl.when(go > 0)(_drain_refill)
            else:
                _drain_refill()
        return 0

    lax.fori_loop(0, MAIN // NBUF, outer_body, 0)

    _wb_wait(NBUF - 1, MAIN - 1)
    _out_wait(NBUF - 1)

    for t in range(NCHUNK - MAIN):
        j = MAIN + t
        base = ebase + j * CHUNK
        pltpu.sync_copy(pe_hbm.at[pl.ds(base, CHUNK), :], ebufs[t])
        pltpu.sync_copy(w_hbm.at[pl.ds(base, CHUNK)], wrs[t])
        pltpu.sync_copy(dst_hbm.at[pl.ds(base, CHUNK)], irs[t])
        pltpu.sync_copy(ebufs[t], peout_hbm.at[pl.ds(base, CHUNK), :])

        @plsc.parallel_loop(0, CHUNK, unroll=4)
        def _rows_tail(r):
            wspl = plsc.load_gather(
                wrs[t], [jnp.broadcast_to(r, (16,)).astype(jnp.int32)])
            eb = ebufs[t]
            ob = obufs[t]
            for d in range(D // 16):
                sl = pl.ds(d * 16, 16)
                ob[r, sl] = eb[r, sl] * wspl

        pltpu.sync_copy(obufs[t], acc.at[irs[t]], add=True)

    if REM > 0:
        rbase = ebase + NCHUNK * CHUNK
        pltpu.sync_copy(pe_hbm.at[pl.ds(rbase, REM), :],
                        ebufs[0].at[pl.ds(0, REM)])
        pltpu.sync_copy(w_hbm.at[pl.ds(rbase, REM)],
                        wrs[0].at[pl.ds(0, REM)])
        pltpu.sync_copy(dst_hbm.at[pl.ds(rbase, REM)], ir_rem)
        pltpu.sync_copy(ebufs[0].at[pl.ds(0, REM)],
                        peout_hbm.at[pl.ds(rbase, REM), :])

        @plsc.parallel_loop(0, REM, unroll=4)
        def _rows_rem(r):
            wspl = plsc.load_gather(
                wrs[0], [jnp.broadcast_to(r, (16,)).astype(jnp.int32)])
            for d in range(D // 16):
                sl = pl.ds(d * 16, 16)
                obufs[0][r, sl] = ebufs[0][r, sl] * wspl

        pltpu.sync_copy(obufs[0].at[pl.ds(0, REM)], acc.at[ir_rem], add=True)

    plsc.subcore_barrier()

    pltpu.sync_copy(acc.at[pl.ds(s * ZR, ZR)],
                    part_hbm.at[c, pl.ds(s * ZR, ZR)])

    @pl.when(s == 0)
    def _read_tail():
        pltpu.sync_copy(acc.at[pl.ds(NS * ZR, ZTAIL)],
                        part_hbm.at[c, pl.ds(NS * ZR, ZTAIL)])


@jax.jit
def _sc_call(pe, w, dst, zeros_nd):
    mesh = plsc.VectorSubcoreMesh(core_axis_name="c", subcore_axis_name="s")
    return pl.kernel(
        _sc_scatter,
        out_type=(
            jax.ShapeDtypeStruct((NC, N, D), jnp.float32),
            jax.ShapeDtypeStruct((E, D), jnp.float32),
        ),
        mesh=mesh,
        compiler_params=pltpu.CompilerParams(needs_layout_passes=False),
        scratch_types=(
            [pltpu.VMEM((CHUNK, D), jnp.float32)] * NBUF
            + [pltpu.VMEM((CHUNK, D), jnp.float32)] * NBUF
            + [pltpu.VMEM((CHUNK,), jnp.float32)] * NBUF
            + [pltpu.VMEM((CHUNK,), jnp.int32)] * NBUF
            + [pltpu.VMEM((max(REM, 8),), jnp.int32)]
            + [pltpu.VMEM_SHARED((N, D), jnp.float32)]
            + [pltpu.SemaphoreType.DMA] * (3 * NBUF)
        ),
    )(pe, w, dst, zeros_nd)


def _combine_body(p0_ref, p1_ref, ns_ref, norm_ref, recall_ref):
    recall = p0_ref[...] + p1_ref[...]
    recall_ref[...] = recall
    norm_ref[...] = recall / ns_ref[...]


@jax.jit
def _combine(parts, ns):
    blk = 5000
    grid = (N // blk,)
    return pl.pallas_call(
        _combine_body,
        grid=grid,
        in_specs=[
            pl.BlockSpec((blk, D), lambda i: (i, 0)),
            pl.BlockSpec((blk, D), lambda i: (i, 0)),
            pl.BlockSpec((blk, 1), lambda i: (i, 0)),
        ],
        out_specs=[
            pl.BlockSpec((blk, D), lambda i: (i, 0)),
            pl.BlockSpec((blk, D), lambda i: (i, 0)),
        ],
        out_shape=[
            jax.ShapeDtypeStruct((N, D), jnp.float32),
            jax.ShapeDtypeStruct((N, D), jnp.float32),
        ],
    )(parts[0], parts[1], ns)


def kernel(poss_edge, weights, neighbours_sum, edges):
    dst = edges[:, 0]
    zeros_nd = jnp.zeros((N, D), jnp.float32)
    parts, pe_out = _sc_call(poss_edge, weights, dst, zeros_nd)
    norm, recall = _combine(parts, neighbours_sum)
    return (norm, pe_out, recall)
